# baseline (device time: 30908 ns/iter reference)
import jax
import jax.numpy as jnp
from jax import lax
from jax.experimental import pallas as pl
from jax.experimental.pallas import tpu as pltpu

N_GLOBAL = 2048
EPS = 1e-5
C = 512


def kernel(x, gamma, beta):
    m, n_loc = x.shape
    nc = m // C

    def body(x_hbm, gamma_ref, beta_ref, out_hbm, x_vmem,
             send_buf, recv_buf, in_sems, out_sems, send_sems, recv_sems):
        my_x = lax.axis_index("x")
        my_y = lax.axis_index("y")
        peer = (my_x, 1 - my_y)

        barrier_sem = pltpu.get_barrier_semaphore()
        pl.semaphore_signal(
            barrier_sem, inc=1, device_id=peer,
            device_id_type=pl.DeviceIdType.MESH,
        )
        pl.semaphore_wait(barrier_sem, 1)

        copies_in = []
        for i in range(nc):
            cp = pltpu.make_async_copy(
                x_hbm.at[pl.ds(i * C, C), :],
                x_vmem.at[pl.ds(i * C, C), :],
                in_sems.at[i],
            )
            cp.start()
            copies_in.append(cp)

        rdmas = []
        for i in range(nc):
            copies_in[i].wait()
            xc = x_vmem[pl.ds(i * C, C), :]
            s1 = jnp.sum(xc, axis=1)
            s2 = jnp.sum(xc * xc, axis=1)
            send_buf[0:1, pl.ds(i * C, C)] = s1.reshape(1, C)
            send_buf[1:2, pl.ds(i * C, C)] = s2.reshape(1, C)
            rdma = pltpu.make_async_remote_copy(
                src_ref=send_buf.at[:, pl.ds(i * C, C)],
                dst_ref=recv_buf.at[:, pl.ds(i * C, C)],
                send_sem=send_sems.at[i],
                recv_sem=recv_sems.at[i],
                device_id=peer,
                device_id_type=pl.DeviceIdType.MESH,
            )
            rdma.start()
            rdmas.append(rdma)

        copies_out = []
        for i in range(nc):
            rdmas[i].wait_recv()
            ds = pl.ds(i * C, C)
            tot1 = send_buf[0:1, ds] + recv_buf[0:1, ds]
            tot2 = send_buf[1:2, ds] + recv_buf[1:2, ds]
            mean_r = tot1 / N_GLOBAL
            var_r = tot2 / N_GLOBAL - mean_r * mean_r
            rstd_r = lax.rsqrt(var_r + EPS)
            mean_c = mean_r.reshape(C, 1)
            rstd_c = rstd_r.reshape(C, 1)
            x_vmem[ds, :] = (
                (x_vmem[ds, :] - mean_c) * rstd_c * gamma_ref[0:1, :]
                + beta_ref[0:1, :]
            )
            cp = pltpu.make_async_copy(
                x_vmem.at[ds, :], out_hbm.at[ds, :], out_sems.at[i],
            )
            cp.start()
            copies_out.append(cp)

        for i in range(nc):
            rdmas[i].wait_send()
            copies_out[i].wait()

    return pl.pallas_call(
        body,
        out_shape=jax.ShapeDtypeStruct((m, n_loc), jnp.float32),
        in_specs=[
            pl.BlockSpec(memory_space=pl.ANY),
            pl.BlockSpec(memory_space=pltpu.VMEM),
            pl.BlockSpec(memory_space=pltpu.VMEM),
        ],
        out_specs=pl.BlockSpec(memory_space=pl.ANY),
        scratch_shapes=[
            pltpu.VMEM((m, n_loc), jnp.float32),
            pltpu.VMEM((2, m), jnp.float32),
            pltpu.VMEM((2, m), jnp.float32),
            pltpu.SemaphoreType.DMA((nc,)),
            pltpu.SemaphoreType.DMA((nc,)),
            pltpu.SemaphoreType.DMA((nc,)),
            pltpu.SemaphoreType.DMA((nc,)),
        ],
        compiler_params=pltpu.CompilerParams(
            collective_id=0, vmem_limit_bytes=48 * 1024 * 1024,
        ),
    )(x, gamma.reshape(1, n_loc), beta.reshape(1, n_loc))
